# baseline (speedup 1.0000x reference)
import functools

import jax
import jax.numpy as jnp
from jax import lax
from jax.experimental import pallas as pl
from jax.experimental.pallas import tpu as pltpu
from jax.experimental.pallas import tpu_sc as plsc

BATCH = 16384
DIM = 32
LANES = 16
NUM_WORKERS = 32
B_PER_W = BATCH // NUM_WORKERS
CHUNK = 128
N_CHUNKS = B_PER_W // CHUNK
SUBCORES = 16


def _dot_kernel(gidx_hbm, sidx_hbm, wg_hbm, ws_hbm, out_hbm,
                gidx_v, sidx_v, grows_v, srows_v, tbuf_v, out_v,
                gsp_sh, ssp_sh, sem_g, sem_s, sem_c):
    cid = lax.axis_index("c")
    sid = lax.axis_index("s")
    wid = sid * 2 + cid
    base = wid * B_PER_W
    srow0 = sid * CHUNK

    pltpu.sync_copy(gidx_hbm.at[pl.ds(base, B_PER_W)], gidx_v)
    pltpu.sync_copy(sidx_hbm.at[pl.ds(base, B_PER_W)], sidx_v)

    lane = lax.iota(jnp.int32, LANES)
    lo = pl.ds(0, LANES)
    hi = pl.ds(LANES, LANES)

    for c in range(N_CHUNKS):
        def issue(k, carry):
            giv = gidx_v[pl.ds(c * CHUNK + k * LANES, LANES)]
            siv = sidx_v[pl.ds(c * CHUNK + k * LANES, LANES)]
            for l in range(LANES):
                dst = pl.ds(srow0 + k * LANES + l, 1)
                pltpu.async_copy(
                    wg_hbm.at[pl.ds(giv[l], 1), :], gsp_sh.at[dst, :], sem_g)
                pltpu.async_copy(
                    ws_hbm.at[pl.ds(siv[l], 1), :], ssp_sh.at[dst, :], sem_s)
            return carry

        lax.fori_loop(0, CHUNK // LANES, issue, 0)
        pltpu.make_async_copy(
            wg_hbm.at[pl.ds(0, CHUNK), :],
            gsp_sh.at[pl.ds(srow0, CHUNK), :], sem_g).wait()
        pltpu.make_async_copy(
            ws_hbm.at[pl.ds(0, CHUNK), :],
            ssp_sh.at[pl.ds(srow0, CHUNK), :], sem_s).wait()

        cp1 = pltpu.async_copy(gsp_sh.at[pl.ds(srow0, CHUNK), :], grows_v,
                               sem_c)
        cp2 = pltpu.async_copy(ssp_sh.at[pl.ds(srow0, CHUNK), :], srows_v,
                               sem_c)
        cp1.wait()
        cp2.wait()

        def group(k, carry):
            for j in range(LANES):
                r = k * LANES + j
                t = (grows_v[r, lo] * srows_v[r, lo]
                     + grows_v[r, hi] * srows_v[r, hi])
                plsc.store_scatter(tbuf_v, [lane * LANES + j], t)
            acc = tbuf_v[pl.ds(0, LANES)]
            for kk in range(1, LANES):
                acc = acc + tbuf_v[pl.ds(kk * LANES, LANES)]
            out_v[pl.ds(c * CHUNK + k * LANES, LANES)] = acc
            return carry

        lax.fori_loop(0, CHUNK // LANES, group, 0)

    pltpu.sync_copy(out_v, out_hbm.at[pl.ds(base, B_PER_W)])


@jax.jit
def _run(gene_indices, spot_indices, W_genes, W_spots):
    k = functools.partial(
        pl.kernel,
        mesh=plsc.VectorSubcoreMesh(core_axis_name="c", subcore_axis_name="s"),
        out_type=jax.ShapeDtypeStruct((BATCH,), jnp.float32),
        compiler_params=pltpu.CompilerParams(
            needs_layout_passes=False, use_tc_tiling_on_sc=True),
        scratch_types=[
            pltpu.VMEM((B_PER_W,), jnp.int32),
            pltpu.VMEM((B_PER_W,), jnp.int32),
            pltpu.VMEM((CHUNK, DIM), jnp.float32),
            pltpu.VMEM((CHUNK, DIM), jnp.float32),
            pltpu.VMEM((LANES * LANES,), jnp.float32),
            pltpu.VMEM((B_PER_W,), jnp.float32),
            pltpu.VMEM_SHARED((SUBCORES * CHUNK, DIM), jnp.float32),
            pltpu.VMEM_SHARED((SUBCORES * CHUNK, DIM), jnp.float32),
            pltpu.SemaphoreType.DMA,
            pltpu.SemaphoreType.DMA,
            pltpu.SemaphoreType.DMA,
        ],
    )(_dot_kernel)
    return k(gene_indices, spot_indices, W_genes, W_spots)


def kernel(gene_indices, spot_indices, W_genes, W_spots):
    return _run(gene_indices.astype(jnp.int32),
                spot_indices.astype(jnp.int32),
                W_genes, W_spots)


# speedup vs baseline: 1.1051x; 1.1051x over previous
import functools

import jax
import jax.numpy as jnp
from jax import lax
from jax.experimental import pallas as pl
from jax.experimental.pallas import tpu as pltpu
from jax.experimental.pallas import tpu_sc as plsc

BATCH = 16384
DIM = 32
LANES = 16
NUM_WORKERS = 32
B_PER_W = BATCH // NUM_WORKERS
CHUNK = 256
N_CHUNKS = B_PER_W // CHUNK


def _dot_kernel(gidx_hbm, sidx_hbm, wg_hbm, ws_hbm, out_hbm,
                gidx_v, sidx_v, grows_v, srows_v, tbuf_v, out_v,
                sem_g, sem_s):
    wid = lax.axis_index("s") * 2 + lax.axis_index("c")
    base = wid * B_PER_W

    pltpu.sync_copy(gidx_hbm.at[pl.ds(base, B_PER_W)], gidx_v)
    pltpu.sync_copy(sidx_hbm.at[pl.ds(base, B_PER_W)], sidx_v)

    lane = lax.iota(jnp.int32, LANES)
    lo = pl.ds(0, LANES)
    hi = pl.ds(LANES, LANES)

    for c in range(N_CHUNKS):
        def issue(k, carry):
            giv = gidx_v[pl.ds(c * CHUNK + k * LANES, LANES)]
            siv = sidx_v[pl.ds(c * CHUNK + k * LANES, LANES)]
            for l in range(LANES):
                dst = pl.ds(k * LANES + l, 1)
                pltpu.async_copy(
                    wg_hbm.at[pl.ds(giv[l], 1), :], grows_v.at[dst, :], sem_g)
                pltpu.async_copy(
                    ws_hbm.at[pl.ds(siv[l], 1), :], srows_v.at[dst, :], sem_s)
            return carry

        lax.fori_loop(0, CHUNK // LANES, issue, 0)
        pltpu.make_async_copy(
            wg_hbm.at[pl.ds(0, CHUNK), :], grows_v, sem_g).wait()
        pltpu.make_async_copy(
            ws_hbm.at[pl.ds(0, CHUNK), :], srows_v, sem_s).wait()

        def group(k, carry):
            for j in range(LANES):
                r = k * LANES + j
                t = (grows_v[r, lo] * srows_v[r, lo]
                     + grows_v[r, hi] * srows_v[r, hi])
                plsc.store_scatter(tbuf_v, [lane * LANES + j], t)
            acc = tbuf_v[pl.ds(0, LANES)]
            for kk in range(1, LANES):
                acc = acc + tbuf_v[pl.ds(kk * LANES, LANES)]
            out_v[pl.ds(c * CHUNK + k * LANES, LANES)] = acc
            return carry

        lax.fori_loop(0, CHUNK // LANES, group, 0)

    pltpu.sync_copy(out_v, out_hbm.at[pl.ds(base, B_PER_W)])


@jax.jit
def _run(gene_indices, spot_indices, W_genes, W_spots):
    k = functools.partial(
        pl.kernel,
        mesh=plsc.VectorSubcoreMesh(core_axis_name="c", subcore_axis_name="s"),
        out_type=jax.ShapeDtypeStruct((BATCH,), jnp.float32),
        compiler_params=pltpu.CompilerParams(
            needs_layout_passes=False, use_tc_tiling_on_sc=True),
        scratch_types=[
            pltpu.VMEM((B_PER_W,), jnp.int32),
            pltpu.VMEM((B_PER_W,), jnp.int32),
            pltpu.VMEM((CHUNK, DIM), jnp.float32),
            pltpu.VMEM((CHUNK, DIM), jnp.float32),
            pltpu.VMEM((LANES * LANES,), jnp.float32),
            pltpu.VMEM((B_PER_W,), jnp.float32),
            pltpu.SemaphoreType.DMA,
            pltpu.SemaphoreType.DMA,
        ],
    )(_dot_kernel)
    return k(gene_indices, spot_indices, W_genes, W_spots)


def kernel(gene_indices, spot_indices, W_genes, W_spots):
    return _run(gene_indices.astype(jnp.int32),
                spot_indices.astype(jnp.int32),
                W_genes, W_spots)
